# baseline (speedup 1.0000x reference)
import functools

import jax
import jax.numpy as jnp
from jax import lax
from jax.experimental import pallas as pl
from jax.experimental.pallas import tpu as pltpu
from jax.experimental.pallas import tpu_sc as plsc

B = 16384
D = 64
OFF = 500
STAGE_BASE = 496
STAGE_ROWS = 512
OUT_CHUNK = 128


@functools.cache
def _build():
    info = plsc.get_sparse_core_info()
    nc, ns, nl = info.num_cores, info.num_subcores, info.num_lanes
    nw = nc * ns
    b_per_w = B // nw
    nblk = b_per_w // nl
    blk_per_chunk = OUT_CHUNK // nl

    mesh = plsc.VectorSubcoreMesh(core_axis_name="c", subcore_axis_name="s")

    @functools.partial(
        pl.kernel,
        mesh=mesh,
        out_type=jax.ShapeDtypeStruct((B // 2, 2 * D), jnp.float32),
        scratch_types=[
            pltpu.VMEM((STAGE_ROWS * D,), jnp.float32),
            pltpu.VMEM((b_per_w,), jnp.int32),
            pltpu.VMEM((b_per_w // 2, 2 * D), jnp.float32),
            pltpu.SemaphoreType.DMA,
            pltpu.SemaphoreType.DMA,
        ],
    )
    def gather_kernel(tab_hbm, idx_hbm, out_hbm, tab_v, idx_v, rows_v, sem, osem):
        wid = lax.axis_index("s") * nc + lax.axis_index("c")
        tab_cp = pltpu.async_copy(tab_hbm, tab_v, sem)
        pltpu.sync_copy(idx_hbm.at[pl.ds(wid * b_per_w, b_per_w)], idx_v)
        tab_cp.wait()

        def fill_block(k):
            base = k * nl
            v = idx_v[pl.ds(base, nl)]
            pair_base = k * (nl // 2)
            for i in range(nl):
                src0 = (v[i] + (OFF - STAGE_BASE)) * D
                half = (i % 2) * D
                for c in range(0, D, nl):
                    rows_v[pair_base + i // 2, pl.ds(half + c, nl)] = tab_v[
                        pl.ds(src0 + c, nl)
                    ]

        out_cps = []
        for q in range(b_per_w // OUT_CHUNK):
            def body(k, carry):
                fill_block(q * blk_per_chunk + k)
                return carry

            lax.fori_loop(0, blk_per_chunk, body, 0)
            out_cps.append(
                pltpu.async_copy(
                    rows_v.at[pl.ds(q * OUT_CHUNK // 2, OUT_CHUNK // 2)],
                    out_hbm.at[
                        pl.ds(
                            pl.multiple_of(
                                (wid * b_per_w + q * OUT_CHUNK) // 2, 8
                            ),
                            OUT_CHUNK // 2,
                        )
                    ],
                    osem,
                )
            )
        for cp in out_cps:
            cp.wait()

    return gather_kernel


def kernel(L_self_modules_embedding_parameters_weight_, L_batch_):
    tab = L_self_modules_embedding_parameters_weight_[
        STAGE_BASE : STAGE_BASE + STAGE_ROWS
    ].reshape(-1)
    idx = L_batch_.astype(jnp.int32)
    out = _build()(tab, idx)
    return (out.reshape(B, D),)


# speedup vs baseline: 1.0472x; 1.0472x over previous
import functools

import jax
import jax.numpy as jnp
from jax import lax
from jax.experimental import pallas as pl
from jax.experimental.pallas import tpu as pltpu
from jax.experimental.pallas import tpu_sc as plsc

B = 16384
D = 64
OFF = 500
STAGE_BASE = 496
STAGE_ROWS = 512
OUT_CHUNK = 128
UNROLL = 4


@functools.cache
def _build():
    info = plsc.get_sparse_core_info()
    nc, ns, nl = info.num_cores, info.num_subcores, info.num_lanes
    nw = nc * ns
    b_per_w = B // nw
    blk_per_chunk = OUT_CHUNK // nl

    mesh = plsc.VectorSubcoreMesh(core_axis_name="c", subcore_axis_name="s")

    @functools.partial(
        pl.kernel,
        mesh=mesh,
        out_type=jax.ShapeDtypeStruct((B, D), jnp.float32),
        scratch_types=[
            pltpu.VMEM((STAGE_ROWS * D,), jnp.float32),
            pltpu.VMEM((b_per_w,), jnp.int32),
            pltpu.VMEM((b_per_w, D), jnp.float32),
            pltpu.SemaphoreType.DMA,
            pltpu.SemaphoreType.DMA,
        ],
    )
    def gather_kernel(tab_hbm, idx_hbm, out_hbm, tab_v, idx_v, rows_v, sem, osem):
        wid = lax.axis_index("s") * nc + lax.axis_index("c")
        tab_cp = pltpu.async_copy(tab_hbm, tab_v, sem)
        pltpu.sync_copy(idx_hbm.at[pl.ds(wid * b_per_w, b_per_w)], idx_v)
        tab_cp.wait()

        def fill_block(k):
            base = k * nl
            v = idx_v[pl.ds(base, nl)]
            for i in range(nl):
                src0 = (v[i] + (OFF - STAGE_BASE)) * D
                for c in range(0, D, nl):
                    rows_v[base + i, pl.ds(c, nl)] = tab_v[pl.ds(src0 + c, nl)]

        out_cps = []
        for q in range(b_per_w // OUT_CHUNK):
            def body(k, carry):
                fill_block(q * blk_per_chunk + k)
                return carry

            lax.fori_loop(0, blk_per_chunk, body, 0, unroll=UNROLL)
            out_cps.append(
                pltpu.async_copy(
                    rows_v.at[pl.ds(q * OUT_CHUNK, OUT_CHUNK)],
                    out_hbm.at[pl.ds(wid * b_per_w + q * OUT_CHUNK, OUT_CHUNK)],
                    osem,
                )
            )
        for cp in out_cps:
            cp.wait()

    return gather_kernel


def kernel(L_self_modules_embedding_parameters_weight_, L_batch_):
    tab = L_self_modules_embedding_parameters_weight_[
        STAGE_BASE : STAGE_BASE + STAGE_ROWS
    ].reshape(-1)
    idx = L_batch_.astype(jnp.int32)
    out = _build()(tab, idx)
    return (out,)


# speedup vs baseline: 1.1606x; 1.1083x over previous
import functools

import jax
import jax.numpy as jnp
from jax import lax
from jax.experimental import pallas as pl
from jax.experimental.pallas import tpu as pltpu
from jax.experimental.pallas import tpu_sc as plsc

B = 16384
D = 64
OFF = 500
STAGE_BASE = 496
STAGE_ROWS = 512
OUT_CHUNK = 128


@functools.cache
def _build():
    info = plsc.get_sparse_core_info()
    nc, ns, nl = info.num_cores, info.num_subcores, info.num_lanes
    nw = nc * ns
    b_per_w = B // nw
    blk_per_chunk = OUT_CHUNK // nl

    mesh = plsc.VectorSubcoreMesh(core_axis_name="c", subcore_axis_name="s")

    @functools.partial(
        pl.kernel,
        mesh=mesh,
        out_type=jax.ShapeDtypeStruct((B, D), jnp.float32),
        scratch_types=[
            pltpu.VMEM((STAGE_ROWS * D,), jnp.float32),
            pltpu.VMEM((b_per_w,), jnp.int32),
            pltpu.VMEM((b_per_w, D), jnp.float32),
            pltpu.SemaphoreType.DMA,
            pltpu.SemaphoreType.DMA,
        ],
    )
    def gather_kernel(tab_hbm, idx_hbm, out_hbm, tab_v, idx_v, rows_v, sem, osem):
        wid = lax.axis_index("s") * nc + lax.axis_index("c")
        tab_cp = pltpu.async_copy(tab_hbm, tab_v, sem)
        pltpu.sync_copy(idx_hbm.at[pl.ds(wid * b_per_w, b_per_w)], idx_v)
        tab_cp.wait()

        def fill_block(k):
            base = k * nl
            v = idx_v[pl.ds(base, nl)]
            for i in range(nl):
                src0 = (v[i] + (OFF - STAGE_BASE)) * D
                for c in range(0, D, nl):
                    rows_v[base + i, pl.ds(c, nl)] = tab_v[pl.ds(src0 + c, nl)]

        out_cps = []
        for q in range(b_per_w // OUT_CHUNK):
            def body(k, carry):
                fill_block(q * blk_per_chunk + k)
                return carry

            lax.fori_loop(0, blk_per_chunk, body, 0)
            out_cps.append(
                pltpu.async_copy(
                    rows_v.at[pl.ds(q * OUT_CHUNK, OUT_CHUNK)],
                    out_hbm.at[pl.ds(wid * b_per_w + q * OUT_CHUNK, OUT_CHUNK)],
                    osem,
                )
            )
        for cp in out_cps:
            cp.wait()

    return gather_kernel


def kernel(L_self_modules_embedding_parameters_weight_, L_batch_):
    tab = L_self_modules_embedding_parameters_weight_[
        STAGE_BASE : STAGE_BASE + STAGE_ROWS
    ].reshape(-1)
    idx = L_batch_.astype(jnp.int32)
    out = _build()(tab, idx)
    return (out,)
